# baseline (speedup 1.0000x reference)
import functools

import jax
import jax.numpy as jnp
from jax import lax
from jax.experimental import pallas as pl
from jax.experimental.pallas import tpu as pltpu
from jax.experimental.pallas import tpu_sc as plsc

D_MODEL = 1024
B_TOTAL = 16384
LANES = 16
VECS_PER_ROW = D_MODEL // LANES

_info = plsc.get_sparse_core_info()
NW = _info.num_cores * _info.num_subcores
B_PER_W = B_TOTAL // NW
CHUNK = 64
N_CHUNKS = B_PER_W // CHUNK

_mesh = plsc.VectorSubcoreMesh(core_axis_name="c", subcore_axis_name="s")


@functools.partial(
    pl.kernel,
    mesh=_mesh,
    out_type=jax.ShapeDtypeStruct((B_TOTAL, D_MODEL), jnp.float32),
    scratch_types=[
        pltpu.VMEM((CHUNK,), jnp.int32),
        pltpu.VMEM((CHUNK, D_MODEL), jnp.float32),
        pltpu.VMEM((LANES,), jnp.float32),
        pltpu.SemaphoreType.DMA,
    ],
)
def _gather_scale(table_hbm, idx_hbm, w_hbm, out_hbm, idx_v, rows_v, w_v, sem):
    wid = lax.axis_index("s") * _info.num_cores + lax.axis_index("c")
    base = wid * B_PER_W
    pltpu.sync_copy(w_hbm, w_v)
    wv = w_v[...]

    for c in range(N_CHUNKS):
        row0 = base + c * CHUNK
        pltpu.sync_copy(idx_hbm.at[pl.ds(row0, CHUNK)], idx_v)
        pltpu.async_copy(table_hbm.at[idx_v], rows_v, sem).wait()

        def body(i, carry):
            r = i // VECS_PER_ROW
            j = (i % VECS_PER_ROW) * LANES
            rows_v[r, pl.ds(j, LANES)] = rows_v[r, pl.ds(j, LANES)] * wv
            return carry

        lax.fori_loop(0, CHUNK * VECS_PER_ROW, body, 0)
        pltpu.sync_copy(rows_v, out_hbm.at[pl.ds(row0, CHUNK)])


def kernel(pos_ids, weight, emb):
    idx = pos_ids.reshape(-1).astype(jnp.int32)
    w16 = jnp.broadcast_to(weight.astype(jnp.float32), (LANES,))
    out = _gather_scale(emb, idx, w16)
    return out.reshape(pos_ids.shape + (D_MODEL,))


# speedup vs baseline: 3.3497x; 3.3497x over previous
import functools

import jax
import jax.numpy as jnp
from jax import lax
from jax.experimental import pallas as pl
from jax.experimental.pallas import tpu as pltpu
from jax.experimental.pallas import tpu_sc as plsc

D_MODEL = 1024
B_TOTAL = 16384
LANES = 16
VECS_PER_ROW = D_MODEL // LANES

_info = plsc.get_sparse_core_info()
NW = _info.num_cores * _info.num_subcores
B_PER_W = B_TOTAL // NW
CHUNK = 32
N_CHUNKS = B_PER_W // CHUNK

_mesh = plsc.VectorSubcoreMesh(core_axis_name="c", subcore_axis_name="s")


@functools.partial(
    pl.kernel,
    mesh=_mesh,
    out_type=jax.ShapeDtypeStruct((B_TOTAL, D_MODEL), jnp.float32),
    scratch_types=[
        pltpu.VMEM((CHUNK,), jnp.int32),
        pltpu.VMEM((CHUNK,), jnp.int32),
        pltpu.VMEM((CHUNK, D_MODEL), jnp.float32),
        pltpu.VMEM((CHUNK, D_MODEL), jnp.float32),
        pltpu.VMEM((LANES,), jnp.float32),
        pltpu.SemaphoreType.DMA,
        pltpu.SemaphoreType.DMA,
        pltpu.SemaphoreType.DMA,
        pltpu.SemaphoreType.DMA,
    ],
)
def _gather_scale(table_hbm, idx_hbm, w_hbm, out_hbm,
                  idx0, idx1, rows0, rows1, w_v,
                  gsem0, gsem1, ssem0, ssem1):
    wid = lax.axis_index("s") * _info.num_cores + lax.axis_index("c")
    base = wid * B_PER_W
    pltpu.sync_copy(w_hbm, w_v)
    wv = w_v[...]

    idx_bufs = (idx0, idx1)
    row_bufs = (rows0, rows1)
    gsems = (gsem0, gsem1)
    ssems = (ssem0, ssem1)

    gathers = [None] * N_CHUNKS
    stores = [None] * N_CHUNKS

    pltpu.sync_copy(idx_hbm.at[pl.ds(base, CHUNK)], idx_bufs[0])
    gathers[0] = pltpu.async_copy(table_hbm.at[idx_bufs[0]], row_bufs[0], gsems[0])

    for c in range(N_CHUNKS):
        b = c % 2
        nb = (c + 1) % 2
        if c + 1 < N_CHUNKS:
            pltpu.sync_copy(idx_hbm.at[pl.ds(base + (c + 1) * CHUNK, CHUNK)],
                            idx_bufs[nb])
            if c >= 1:
                stores[c - 1].wait()
            gathers[c + 1] = pltpu.async_copy(
                table_hbm.at[idx_bufs[nb]], row_bufs[nb], gsems[nb])
        gathers[c].wait()

        rows = row_bufs[b]

        @plsc.parallel_loop(0, CHUNK * VECS_PER_ROW, unroll=8)
        def _scale(i):
            r = i >> 6
            off = (i & (VECS_PER_ROW - 1)) * LANES
            rows[r, pl.ds(off, LANES)] = rows[r, pl.ds(off, LANES)] * wv

        stores[c] = pltpu.async_copy(
            rows, out_hbm.at[pl.ds(base + c * CHUNK, CHUNK)], ssems[b])

    stores[N_CHUNKS - 2].wait()
    stores[N_CHUNKS - 1].wait()


def kernel(pos_ids, weight, emb):
    idx = pos_ids.reshape(-1).astype(jnp.int32)
    w16 = jnp.broadcast_to(weight.astype(jnp.float32), (LANES,))
    out = _gather_scale(emb, idx, w16)
    return out.reshape(pos_ids.shape + (D_MODEL,))


# speedup vs baseline: 3.3526x; 1.0009x over previous
import functools

import jax
import jax.numpy as jnp
from jax import lax
from jax.experimental import pallas as pl
from jax.experimental.pallas import tpu as pltpu
from jax.experimental.pallas import tpu_sc as plsc

D_MODEL = 1024
B_TOTAL = 16384
LANES = 16
VECS_PER_ROW = D_MODEL // LANES

_info = plsc.get_sparse_core_info()
NW = _info.num_cores * _info.num_subcores
B_PER_W = B_TOTAL // NW
CHUNK = 32
N_CHUNKS = B_PER_W // CHUNK

_mesh = plsc.VectorSubcoreMesh(core_axis_name="c", subcore_axis_name="s")


W_PER_ROW = 4096 // B_PER_W


@functools.partial(
    pl.kernel,
    mesh=_mesh,
    out_type=jax.ShapeDtypeStruct((B_TOTAL, D_MODEL), jnp.float32),
    scratch_types=[
        pltpu.VMEM((B_PER_W,), jnp.int32),
        pltpu.VMEM((CHUNK, D_MODEL), jnp.float32),
        pltpu.VMEM((CHUNK, D_MODEL), jnp.float32),
        pltpu.VMEM((LANES,), jnp.float32),
        pltpu.SemaphoreType.DMA,
        pltpu.SemaphoreType.DMA,
        pltpu.SemaphoreType.DMA,
        pltpu.SemaphoreType.DMA,
    ],
)
def _gather_scale(table_hbm, idx_hbm, w_hbm, out_hbm,
                  idx_all, rows0, rows1, w_v,
                  gsem0, gsem1, ssem0, ssem1):
    wid = lax.axis_index("s") * _info.num_cores + lax.axis_index("c")
    base = wid * B_PER_W
    pltpu.sync_copy(
        idx_hbm.at[wid // W_PER_ROW, pl.ds((wid % W_PER_ROW) * B_PER_W, B_PER_W)],
        idx_all)
    pltpu.sync_copy(w_hbm, w_v)
    wv = w_v[...]

    row_bufs = (rows0, rows1)
    gsems = (gsem0, gsem1)
    ssems = (ssem0, ssem1)

    gathers = [None] * N_CHUNKS
    stores = [None] * N_CHUNKS

    gathers[0] = pltpu.async_copy(
        table_hbm.at[idx_all.at[pl.ds(0, CHUNK)]], row_bufs[0], gsems[0])

    for c in range(N_CHUNKS):
        b = c % 2
        nb = (c + 1) % 2
        if c + 1 < N_CHUNKS:
            if c >= 1:
                stores[c - 1].wait()
            gathers[c + 1] = pltpu.async_copy(
                table_hbm.at[idx_all.at[pl.ds((c + 1) * CHUNK, CHUNK)]],
                row_bufs[nb], gsems[nb])
        gathers[c].wait()

        rows = row_bufs[b]

        @plsc.parallel_loop(0, CHUNK * VECS_PER_ROW, unroll=8)
        def _scale(i):
            r = i >> 6
            off = (i & (VECS_PER_ROW - 1)) * LANES
            rows[r, pl.ds(off, LANES)] = rows[r, pl.ds(off, LANES)] * wv

        stores[c] = pltpu.async_copy(
            rows, out_hbm.at[pl.ds(base + c * CHUNK, CHUNK)], ssems[b])

    stores[N_CHUNKS - 2].wait()
    stores[N_CHUNKS - 1].wait()


def kernel(pos_ids, weight, emb):
    w16 = jnp.broadcast_to(weight.astype(jnp.float32), (LANES,))
    out = _gather_scale(emb, pos_ids, w16)
    return out.reshape(pos_ids.shape + (D_MODEL,))


# speedup vs baseline: 3.3664x; 1.0041x over previous
import functools

import jax
import jax.numpy as jnp
from jax import lax
from jax.experimental import pallas as pl
from jax.experimental.pallas import tpu as pltpu
from jax.experimental.pallas import tpu_sc as plsc

D_MODEL = 1024
B_TOTAL = 16384
LANES = 16
VECS_PER_ROW = D_MODEL // LANES

_info = plsc.get_sparse_core_info()
NW = _info.num_cores * _info.num_subcores
B_PER_W = B_TOTAL // NW
CHUNK = 32
N_CHUNKS = B_PER_W // CHUNK

_mesh = plsc.VectorSubcoreMesh(core_axis_name="c", subcore_axis_name="s")


W_PER_ROW = 4096 // B_PER_W


@functools.partial(
    pl.kernel,
    mesh=_mesh,
    out_type=jax.ShapeDtypeStruct((B_TOTAL, D_MODEL), jnp.float32),
    scratch_types=[
        pltpu.VMEM((B_PER_W,), jnp.int32),
        pltpu.VMEM((CHUNK, D_MODEL), jnp.float32),
        pltpu.VMEM((CHUNK, D_MODEL), jnp.float32),
        pltpu.VMEM((CHUNK, D_MODEL), jnp.float32),
        pltpu.VMEM((LANES,), jnp.float32),
        pltpu.SemaphoreType.DMA,
        pltpu.SemaphoreType.DMA,
        pltpu.SemaphoreType.DMA,
        pltpu.SemaphoreType.DMA,
        pltpu.SemaphoreType.DMA,
        pltpu.SemaphoreType.DMA,
    ],
)
def _gather_scale(table_hbm, idx_hbm, w_hbm, out_hbm,
                  idx_all, rows0, rows1, rows2, w_v,
                  gsem0, gsem1, gsem2, ssem0, ssem1, ssem2):
    wid = lax.axis_index("s") * _info.num_cores + lax.axis_index("c")
    base = wid * B_PER_W
    pltpu.sync_copy(
        idx_hbm.at[wid // W_PER_ROW, pl.ds((wid % W_PER_ROW) * B_PER_W, B_PER_W)],
        idx_all)
    pltpu.sync_copy(w_hbm, w_v)
    wv = w_v[...]

    NBUF = 3
    row_bufs = (rows0, rows1, rows2)
    gsems = (gsem0, gsem1, gsem2)
    ssems = (ssem0, ssem1, ssem2)

    gathers = [None] * N_CHUNKS
    stores = [None] * N_CHUNKS

    LEAD = 2
    for c in range(LEAD):
        gathers[c] = pltpu.async_copy(
            table_hbm.at[idx_all.at[pl.ds(c * CHUNK, CHUNK)]],
            row_bufs[c], gsems[c])

    for c in range(N_CHUNKS):
        b = c % NBUF
        gathers[c].wait()
        rows = row_bufs[b]

        @plsc.parallel_loop(0, CHUNK * VECS_PER_ROW, unroll=8)
        def _scale(i):
            r = i >> 6
            off = (i & (VECS_PER_ROW - 1)) * LANES
            rows[r, pl.ds(off, LANES)] = rows[r, pl.ds(off, LANES)] * wv

        stores[c] = pltpu.async_copy(
            rows, out_hbm.at[pl.ds(base + c * CHUNK, CHUNK)], ssems[b])

        nc = c + LEAD
        if nc < N_CHUNKS:
            if nc - NBUF >= 0:
                stores[nc - NBUF].wait()
            gathers[nc] = pltpu.async_copy(
                table_hbm.at[idx_all.at[pl.ds(nc * CHUNK, CHUNK)]],
                row_bufs[nc % NBUF], gsems[nc % NBUF])

    for c in range(N_CHUNKS - NBUF, N_CHUNKS):
        if c >= 0:
            stores[c].wait()


def kernel(pos_ids, weight, emb):
    w16 = jnp.broadcast_to(weight.astype(jnp.float32), (LANES,))
    out = _gather_scale(emb, pos_ids, w16)
    return out.reshape(pos_ids.shape + (D_MODEL,))


# speedup vs baseline: 3.4741x; 1.0320x over previous
import functools

import jax
import jax.numpy as jnp
from jax import lax
from jax.experimental import pallas as pl
from jax.experimental.pallas import tpu as pltpu
from jax.experimental.pallas import tpu_sc as plsc

D_MODEL = 1024
B_TOTAL = 16384
LANES = 16
VECS_PER_ROW = D_MODEL // LANES

_info = plsc.get_sparse_core_info()
NW = _info.num_cores * _info.num_subcores
B_PER_W = B_TOTAL // NW
CHUNK = 32
N_CHUNKS = B_PER_W // CHUNK
W_PER_ROW = 4096 // B_PER_W

_mesh = plsc.VectorSubcoreMesh(core_axis_name="c", subcore_axis_name="s")


@functools.partial(
    pl.kernel,
    mesh=_mesh,
    out_type=jax.ShapeDtypeStruct((B_TOTAL, D_MODEL), jnp.float32),
    scratch_types=[
        pltpu.VMEM((B_PER_W,), jnp.int32),
        pltpu.VMEM((CHUNK, D_MODEL), jnp.float32),
        pltpu.VMEM((CHUNK, D_MODEL), jnp.float32),
        pltpu.VMEM((LANES,), jnp.float32),
        pltpu.SemaphoreType.DMA,
        pltpu.SemaphoreType.DMA,
        pltpu.SemaphoreType.DMA,
        pltpu.SemaphoreType.DMA,
    ],
)
def _gather_scale(table_hbm, idx_hbm, w_hbm, out_hbm,
                  idx_all, rows0, rows1, w_v,
                  gsem0, gsem1, ssem0, ssem1):
    wid = lax.axis_index("s") * _info.num_cores + lax.axis_index("c")
    base = wid * B_PER_W
    pltpu.sync_copy(
        idx_hbm.at[wid // W_PER_ROW, pl.ds((wid % W_PER_ROW) * B_PER_W, B_PER_W)],
        idx_all)
    pltpu.sync_copy(w_hbm, w_v)
    wv = w_v[...]

    row_bufs = (rows0, rows1)
    gsems = (gsem0, gsem1)
    ssems = (ssem0, ssem1)

    def fire_gather(g, b):
        return pltpu.async_copy(
            table_hbm.at[idx_all.at[pl.ds(g * CHUNK, CHUNK)]],
            row_bufs[b], gsems[b])

    fire_gather(0, 0)

    @pl.loop(0, N_CHUNKS, step=2)
    def _outer(c0):
        for b in range(2):
            g = c0 + b
            nb = 1 - b

            @pl.when(g + 1 < N_CHUNKS)
            def _fire_next():
                @pl.when(g >= 1)
                def _drain_store():
                    pltpu.make_async_copy(
                        row_bufs[nb], out_hbm.at[pl.ds(base, CHUNK)],
                        ssems[nb]).wait()
                fire_gather(g + 1, nb)

            pltpu.make_async_copy(
                table_hbm.at[idx_all.at[pl.ds(0, CHUNK)]],
                row_bufs[b], gsems[b]).wait()

            rows = row_bufs[b]

            @plsc.parallel_loop(0, CHUNK * VECS_PER_ROW, unroll=8)
            def _scale(i):
                r = i >> 6
                off = (i & (VECS_PER_ROW - 1)) * LANES
                rows[r, pl.ds(off, LANES)] = rows[r, pl.ds(off, LANES)] * wv

            pltpu.async_copy(
                rows, out_hbm.at[pl.ds(base + g * CHUNK, CHUNK)], ssems[b])

    for b in range(2):
        pltpu.make_async_copy(
            row_bufs[b], out_hbm.at[pl.ds(base, CHUNK)], ssems[b]).wait()


def kernel(pos_ids, weight, emb):
    w16 = jnp.broadcast_to(weight.astype(jnp.float32), (LANES,))
    out = _gather_scale(emb, pos_ids, w16)
    return out.reshape(pos_ids.shape + (D_MODEL,))
